# baseline (speedup 1.0000x reference)
import functools

import jax
import jax.numpy as jnp
import numpy as np
from jax import lax
from jax.experimental import pallas as pl
from jax.experimental.pallas import tpu as pltpu
from jax.experimental.pallas import tpu_sc as plsc

_THETA_RES = 16
_PHI_RES = 8
_MAX_SCALE_LOG = 0.3
_L = 16
_NW = 32
_NBUF = 2
_CHUNK = 16384


def _body(theta_hbm, phi_hbm, grid_hbm, out_hbm, table_v, *scr):
    n = theta_hbm.shape[0]
    per_w = n // _NW
    n_chunks = per_w // _CHUNK
    n_blocks = n_chunks // _NBUF

    th_bufs = scr[0:_NBUF]
    ph_bufs = scr[_NBUF:2 * _NBUF]
    o_bufs = scr[2 * _NBUF:3 * _NBUF]
    isems = scr[3 * _NBUF:4 * _NBUF]
    osems = scr[4 * _NBUF:5 * _NBUF]

    cid = lax.axis_index("c")
    sid = lax.axis_index("s")
    wid = sid * 2 + cid
    base = wid * per_w

    pltpu.sync_copy(grid_hbm, table_v)
    for i in range(_THETA_RES * _PHI_RES // _L):
        g = table_v[pl.ds(i * _L, _L)]
        g = jnp.minimum(jnp.maximum(g, -_MAX_SCALE_LOG), _MAX_SCALE_LOG)
        table_v[pl.ds(i * _L, _L)] = jnp.exp(g)

    scale = jnp.float32(_THETA_RES / (2.0 * np.pi))
    t_max = jnp.int32(_THETA_RES - 1)
    p_max = jnp.int32(_PHI_RES - 1)

    def start_in(c, slot):
        off = base + c * _CHUNK
        pltpu.async_copy(theta_hbm.at[pl.ds(off, _CHUNK)], th_bufs[slot],
                         isems[slot])
        pltpu.async_copy(phi_hbm.at[pl.ds(off, _CHUNK)], ph_bufs[slot],
                         isems[slot])

    def wait_in(slot):
        pltpu.make_async_copy(theta_hbm.at[pl.ds(0, _CHUNK)], th_bufs[slot],
                              isems[slot]).wait()
        pltpu.make_async_copy(phi_hbm.at[pl.ds(0, _CHUNK)], ph_bufs[slot],
                              isems[slot]).wait()

    def start_out(c, slot):
        off = base + c * _CHUNK
        pltpu.async_copy(o_bufs[slot], out_hbm.at[pl.ds(off, _CHUNK)],
                         osems[slot])

    def wait_out(slot):
        pltpu.make_async_copy(o_bufs[slot], out_hbm.at[pl.ds(0, _CHUNK)],
                              osems[slot]).wait()

    def compute(slot):
        th_v, ph_v, o_v = th_bufs[slot], ph_bufs[slot], o_bufs[slot]

        @plsc.parallel_loop(0, _CHUNK // _L, unroll=32)
        def _(i):
            t = th_v[pl.ds(i * _L, _L)]
            p = ph_v[pl.ds(i * _L, _L)]
            ti = jnp.minimum((t * scale).astype(jnp.int32), t_max)
            pi_ = jnp.minimum((p * scale).astype(jnp.int32), p_max)
            flat = ti * _PHI_RES + pi_
            o_v[pl.ds(i * _L, _L)] = plsc.load_gather(table_v, [flat])

    for b in range(_NBUF):
        start_in(b, b)
    for c in range(_NBUF):
        wait_in(c)
        compute(c)
        start_out(c, c)
        start_in(c + _NBUF, c)

    def ring(j, _):
        for b in range(_NBUF):
            c = j * _NBUF + b
            wait_out(b)
            wait_in(b)
            compute(b)
            start_out(c, b)
            start_in(c + _NBUF, b)
        return 0

    lax.fori_loop(1, n_blocks - 1, ring, 0)

    for b in range(_NBUF):
        c = (n_blocks - 1) * _NBUF + b
        wait_out(b)
        wait_in(b)
        compute(b)
        start_out(c, b)
    for b in range(_NBUF):
        wait_out(b)


def kernel(theta, phi, grid):
    n = theta.shape[0]
    grid_flat = grid.reshape(-1)
    mesh = plsc.VectorSubcoreMesh(core_axis_name="c", subcore_axis_name="s")
    run = pl.kernel(
        _body,
        out_type=jax.ShapeDtypeStruct((n,), jnp.float32),
        mesh=mesh,
        scratch_types=(
            [pltpu.VMEM((_THETA_RES * _PHI_RES,), jnp.float32)]
            + [pltpu.VMEM((_CHUNK,), jnp.float32) for _ in range(3 * _NBUF)]
            + [pltpu.SemaphoreType.DMA for _ in range(2 * _NBUF)]
        ),
        compiler_params=pltpu.CompilerParams(needs_layout_passes=False),
    )
    return run(theta, phi, grid_flat)


# speedup vs baseline: 2.0337x; 2.0337x over previous
import functools

import jax
import jax.numpy as jnp
import numpy as np
from jax import lax
from jax.experimental import pallas as pl
from jax.experimental.pallas import tpu as pltpu
from jax.experimental.pallas import tpu_sc as plsc

_THETA_RES = 16
_PHI_RES = 8
_MAX_SCALE_LOG = 0.3
_L = 16
_NW = 32
_NBUF = 2
_CHUNK = 16384


def _body(theta_hbm, phi_hbm, grid_hbm, out_hbm, table_v, *scr):
    n = theta_hbm.shape[0]
    per_w = n // _NW
    n_chunks = per_w // _CHUNK
    n_blocks = n_chunks // _NBUF

    th_bufs = scr[0:_NBUF]
    ph_bufs = scr[_NBUF:2 * _NBUF]
    o_bufs = scr[2 * _NBUF:3 * _NBUF]
    isems = scr[3 * _NBUF:4 * _NBUF]
    osems = scr[4 * _NBUF:5 * _NBUF]

    cid = lax.axis_index("c")
    sid = lax.axis_index("s")
    wid = sid * 2 + cid
    base = wid * per_w

    pltpu.sync_copy(grid_hbm, table_v)
    for i in range(_THETA_RES * _PHI_RES // _L):
        g = table_v[pl.ds(i * _L, _L)]
        g = jnp.minimum(jnp.maximum(g, -_MAX_SCALE_LOG), _MAX_SCALE_LOG)
        table_v[pl.ds(i * _L, _L)] = jnp.exp(g)

    scale = jnp.float32(_THETA_RES / (2.0 * np.pi))
    t_max = jnp.int32(_THETA_RES - 1)
    p_max = jnp.int32(_PHI_RES - 1)

    def start_in(c, slot):
        off = base + c * _CHUNK
        pltpu.async_copy(theta_hbm.at[pl.ds(off, _CHUNK)], th_bufs[slot],
                         isems[slot])
        pltpu.async_copy(phi_hbm.at[pl.ds(off, _CHUNK)], ph_bufs[slot],
                         isems[slot])

    def wait_in(slot):
        pltpu.make_async_copy(theta_hbm.at[pl.ds(0, _CHUNK)], th_bufs[slot],
                              isems[slot]).wait()
        pltpu.make_async_copy(phi_hbm.at[pl.ds(0, _CHUNK)], ph_bufs[slot],
                              isems[slot]).wait()

    def start_out(c, slot):
        off = base + c * _CHUNK
        pltpu.async_copy(o_bufs[slot], out_hbm.at[pl.ds(off, _CHUNK)],
                         osems[slot])

    def wait_out(slot):
        pltpu.make_async_copy(o_bufs[slot], out_hbm.at[pl.ds(0, _CHUNK)],
                              osems[slot]).wait()

    def compute(slot):
        th_v, ph_v, o_v = th_bufs[slot], ph_bufs[slot], o_bufs[slot]

        @plsc.parallel_loop(0, _CHUNK // _L, unroll=16)
        def _(i):
            t = th_v[pl.ds(i * _L, _L)]
            p = ph_v[pl.ds(i * _L, _L)]
            ti = jnp.minimum((t * scale).astype(jnp.int32), t_max)
            pi_ = jnp.minimum((p * scale).astype(jnp.int32), p_max)
            flat = ti * _PHI_RES + pi_
            o_v[pl.ds(i * _L, _L)] = plsc.load_gather(table_v, [flat])

    for b in range(_NBUF):
        start_in(b, b)
    for c in range(_NBUF):
        wait_in(c)
        compute(c)
        start_out(c, c)
        start_in(c + _NBUF, c)

    def ring(j, _):
        for b in range(_NBUF):
            c = j * _NBUF + b
            wait_out(b)
            wait_in(b)
            compute(b)
            start_out(c, b)
            start_in(c + _NBUF, b)
        return 0

    lax.fori_loop(1, n_blocks - 1, ring, 0)

    for b in range(_NBUF):
        c = (n_blocks - 1) * _NBUF + b
        wait_out(b)
        wait_in(b)
        compute(b)
        start_out(c, b)
    for b in range(_NBUF):
        wait_out(b)


def kernel(theta, phi, grid):
    n = theta.shape[0]
    grid_flat = grid.reshape(-1)
    mesh = plsc.VectorSubcoreMesh(core_axis_name="c", subcore_axis_name="s")
    run = pl.kernel(
        _body,
        out_type=jax.ShapeDtypeStruct((n,), jnp.float32),
        mesh=mesh,
        scratch_types=(
            [pltpu.VMEM((_THETA_RES * _PHI_RES,), jnp.float32)]
            + [pltpu.VMEM((_CHUNK,), jnp.float32) for _ in range(3 * _NBUF)]
            + [pltpu.SemaphoreType.DMA for _ in range(2 * _NBUF)]
        ),
        compiler_params=pltpu.CompilerParams(needs_layout_passes=False),
    )
    return run(theta, phi, grid_flat)
